# baseline (speedup 1.0000x reference)
import jax
import jax.numpy as jnp
from jax.experimental import pallas as pl
from jax.experimental.pallas import tpu as pltpu

B, IN, Q, M, OUT = 4096, 2048, 6, 512, 16
COEFS = (-3.0, -0.834, -0.248, 0.248, 0.834)

B_T = 128
IN_T = 2048


def _stats_body(x_ref, mean_ref, var_ref):
    xb = x_ref[...]
    s = jnp.sum(xb, axis=0)
    ss = jnp.sum(xb * xb, axis=0)
    mean = s * (1.0 / B)
    mean_ref[0, :] = mean
    var_ref[0, :] = ss * (1.0 / B) - mean * mean


def _stats(x):
    return pl.pallas_call(
        _stats_body,
        grid=(IN // 512,),
        in_specs=[pl.BlockSpec((B, 512), lambda i: (0, i))],
        out_specs=[
            pl.BlockSpec((1, 512), lambda i: (0, i)),
            pl.BlockSpec((1, 512), lambda i: (0, i)),
        ],
        out_shape=[
            jax.ShapeDtypeStruct((1, IN), jnp.float32),
            jax.ShapeDtypeStruct((1, IN), jnp.float32),
        ],
    )(x)


def _main_body(x_ref, scale_ref, off_ref, shq_ref, n1_ref, n2_ref,
               wq_ref, mx_ref, acc_ref, g1_ref, g2_ref):
    ii = pl.program_id(1)
    normed = x_ref[...] * scale_ref[0, :] + off_ref[0, :]
    nb = normed.astype(jnp.bfloat16)
    pacc = jnp.zeros((B_T, OUT), jnp.float32)
    for q in range(Q):
        rq = jnp.maximum(normed - shq_ref[q, :], 0.0)
        mx_ref[q] = rq
        pacc += jnp.dot(rq.astype(jnp.bfloat16), wq_ref[q].astype(jnp.bfloat16),
                        preferred_element_type=jnp.float32)
    rows = jax.lax.broadcasted_iota(jnp.int32, (IN_T, M), 0) + ii * IN_T
    p1 = (rows == n1_ref[...]).astype(jnp.bfloat16)
    p2 = (rows == n2_ref[...]).astype(jnp.bfloat16)
    g1p = jnp.dot(nb, p1, preferred_element_type=jnp.float32)
    g2p = jnp.dot(nb, p2, preferred_element_type=jnp.float32)

    @pl.when(ii == 0)
    def _():
        acc_ref[...] = jnp.zeros_like(acc_ref)
        g1_ref[...] = jnp.zeros_like(g1_ref)
        g2_ref[...] = jnp.zeros_like(g2_ref)

    acc_ref[...] += pacc
    g1_ref[...] += g1p
    g2_ref[...] += g2p


def _main(x, scale, off, shiftq, n1, n2, w_q):
    return pl.pallas_call(
        _main_body,
        grid=(B // B_T, IN // IN_T),
        in_specs=[
            pl.BlockSpec((B_T, IN_T), lambda ib, ii: (ib, ii)),
            pl.BlockSpec((1, IN_T), lambda ib, ii: (0, ii)),
            pl.BlockSpec((1, IN_T), lambda ib, ii: (0, ii)),
            pl.BlockSpec((Q, IN_T), lambda ib, ii: (0, ii)),
            pl.BlockSpec((1, M), lambda ib, ii: (0, 0)),
            pl.BlockSpec((1, M), lambda ib, ii: (0, 0)),
            pl.BlockSpec((Q, IN_T, OUT), lambda ib, ii: (0, ii, 0)),
        ],
        out_specs=[
            pl.BlockSpec((Q, B_T, IN_T), lambda ib, ii: (0, ib, ii)),
            pl.BlockSpec((B_T, OUT), lambda ib, ii: (ib, 0)),
            pl.BlockSpec((B_T, M), lambda ib, ii: (ib, 0)),
            pl.BlockSpec((B_T, M), lambda ib, ii: (ib, 0)),
        ],
        out_shape=[
            jax.ShapeDtypeStruct((Q, B, IN), jnp.float32),
            jax.ShapeDtypeStruct((B, OUT), jnp.float32),
            jax.ShapeDtypeStruct((B, M), jnp.float32),
            jax.ShapeDtypeStruct((B, M), jnp.float32),
        ],
        compiler_params=pltpu.CompilerParams(
            dimension_semantics=("parallel", "arbitrary"),
        ),
    )(x, scale, off, shiftq, n1, n2, w_q)


def _combine_body(g1_ref, g2_ref, s1_ref, s2_ref, acc_ref, wmin_ref, b_ref,
                  minx_ref, out_ref):
    d1 = jnp.maximum(g1_ref[...] - s1_ref[0, :], 0.0)
    d2 = jnp.maximum(g2_ref[...] - s2_ref[0, :], 0.0)
    mn = jnp.minimum(d1, d2)
    minx_ref[...] = mn
    out_ref[...] = (acc_ref[...]
                    + jnp.dot(mn, wmin_ref[...], preferred_element_type=jnp.float32)
                    + b_ref[0, 0])


def _combine(g1, g2, s1, s2, acc, w_min, biases):
    return pl.pallas_call(
        _combine_body,
        grid=(B // B_T,),
        in_specs=[
            pl.BlockSpec((B_T, M), lambda ib: (ib, 0)),
            pl.BlockSpec((B_T, M), lambda ib: (ib, 0)),
            pl.BlockSpec((1, M), lambda ib: (0, 0)),
            pl.BlockSpec((1, M), lambda ib: (0, 0)),
            pl.BlockSpec((B_T, OUT), lambda ib: (ib, 0)),
            pl.BlockSpec((M, OUT), lambda ib: (0, 0)),
            pl.BlockSpec((1, 1), lambda ib: (0, 0)),
        ],
        out_specs=[
            pl.BlockSpec((B_T, M), lambda ib: (ib, 0)),
            pl.BlockSpec((B_T, OUT), lambda ib: (ib, 0)),
        ],
        out_shape=[
            jax.ShapeDtypeStruct((B, M), jnp.float32),
            jax.ShapeDtypeStruct((B, OUT), jnp.float32),
        ],
    )(g1, g2, s1, s2, acc, w_min, biases)


def kernel(x, init_struct, beta, gamma, w, biases, chosen_index):
    mean2, var2 = _stats(x)
    mean, var = mean2[0], var2[0]
    inv = jax.lax.rsqrt(var + 0.001)
    scale = gamma * inv
    off = beta - mean * scale
    shiftq = jnp.concatenate(
        [jnp.zeros((1, IN), jnp.float32)]
        + [((c * var + mean) * gamma - beta)[None, :] for c in COEFS], axis=0)
    n1 = chosen_index[:, 1]
    q1 = chosen_index[:, 2]
    n2 = chosen_index[:, 3]
    q2 = chosen_index[:, 4]
    s1 = shiftq[q1, n1][None, :]
    s2 = shiftq[q2, n2][None, :]
    w_q = jnp.transpose(w[:IN * Q].reshape(IN, Q, OUT), (1, 0, 2))
    mx, acc, g1, g2 = _main(x, scale[None, :], off[None, :], shiftq,
                            n1[None, :], n2[None, :], w_q)
    min_x, output = _combine(g1, g2, s1, s2, acc, w[IN * Q:], biases.reshape(1, 1))
    max_x = jnp.transpose(mx, (1, 2, 0))
    return (output, w, max_x, min_x)


# speedup vs baseline: 1.2052x; 1.2052x over previous
import jax
import jax.numpy as jnp
from jax.experimental import pallas as pl
from jax.experimental.pallas import tpu as pltpu

B, IN, Q, M, OUT = 4096, 2048, 6, 512, 16
COEFS = (-3.0, -0.834, -0.248, 0.248, 0.834)

B_T = 256


def _stats_body(x_ref, mean_ref, var_ref):
    xb = x_ref[...]
    s = jnp.sum(xb, axis=0)
    ss = jnp.sum(xb * xb, axis=0)
    mean = s * (1.0 / B)
    mean_ref[0, :] = mean
    var_ref[0, :] = ss * (1.0 / B) - mean * mean


def _stats(x):
    return pl.pallas_call(
        _stats_body,
        grid=(IN // 512,),
        in_specs=[pl.BlockSpec((B, 512), lambda i: (0, i))],
        out_specs=[
            pl.BlockSpec((1, 512), lambda i: (0, i)),
            pl.BlockSpec((1, 512), lambda i: (0, i)),
        ],
        out_shape=[
            jax.ShapeDtypeStruct((1, IN), jnp.float32),
            jax.ShapeDtypeStruct((1, IN), jnp.float32),
        ],
    )(x)


def _main_body(x_ref, scale_ref, off_ref, shq_ref, n1_ref, n2_ref,
               wq_ref, s1_ref, s2_ref, wmin_ref, b_ref,
               mx_ref, minx_ref, out_ref):
    normed = x_ref[...] * scale_ref[0, :] + off_ref[0, :]
    nb = normed.astype(jnp.bfloat16)
    pacc = jnp.zeros((B_T, OUT), jnp.float32)
    for q in range(Q):
        rq = jnp.maximum(normed - shq_ref[q, :], 0.0)
        mx_ref[q] = rq
        pacc += jnp.dot(rq.astype(jnp.bfloat16), wq_ref[q].astype(jnp.bfloat16),
                        preferred_element_type=jnp.float32)
    rows = jax.lax.broadcasted_iota(jnp.int32, (IN, M), 0)
    p1 = (rows == n1_ref[...]).astype(jnp.bfloat16)
    p2 = (rows == n2_ref[...]).astype(jnp.bfloat16)
    g1 = jnp.dot(nb, p1, preferred_element_type=jnp.float32)
    g2 = jnp.dot(nb, p2, preferred_element_type=jnp.float32)
    d1 = jnp.maximum(g1 - s1_ref[0, :], 0.0)
    d2 = jnp.maximum(g2 - s2_ref[0, :], 0.0)
    mn = jnp.minimum(d1, d2)
    minx_ref[...] = mn
    out_ref[...] = (pacc
                    + jnp.dot(mn, wmin_ref[...], preferred_element_type=jnp.float32)
                    + b_ref[0, 0])


def _main(x, scale, off, shiftq, n1, n2, w_q, s1, s2, w_min, biases):
    return pl.pallas_call(
        _main_body,
        grid=(B // B_T,),
        in_specs=[
            pl.BlockSpec((B_T, IN), lambda ib: (ib, 0)),
            pl.BlockSpec((1, IN), lambda ib: (0, 0)),
            pl.BlockSpec((1, IN), lambda ib: (0, 0)),
            pl.BlockSpec((Q, IN), lambda ib: (0, 0)),
            pl.BlockSpec((1, M), lambda ib: (0, 0)),
            pl.BlockSpec((1, M), lambda ib: (0, 0)),
            pl.BlockSpec((Q, IN, OUT), lambda ib: (0, 0, 0)),
            pl.BlockSpec((1, M), lambda ib: (0, 0)),
            pl.BlockSpec((1, M), lambda ib: (0, 0)),
            pl.BlockSpec((M, OUT), lambda ib: (0, 0)),
            pl.BlockSpec((1, 1), lambda ib: (0, 0)),
        ],
        out_specs=[
            pl.BlockSpec((Q, B_T, IN), lambda ib: (0, ib, 0)),
            pl.BlockSpec((B_T, M), lambda ib: (ib, 0)),
            pl.BlockSpec((B_T, OUT), lambda ib: (ib, 0)),
        ],
        out_shape=[
            jax.ShapeDtypeStruct((Q, B, IN), jnp.float32),
            jax.ShapeDtypeStruct((B, M), jnp.float32),
            jax.ShapeDtypeStruct((B, OUT), jnp.float32),
        ],
        compiler_params=pltpu.CompilerParams(
            dimension_semantics=("arbitrary",),
        ),
    )(x, scale, off, shiftq, n1, n2, w_q, s1, s2, w_min, biases)


def kernel(x, init_struct, beta, gamma, w, biases, chosen_index):
    mean2, var2 = _stats(x)
    mean, var = mean2[0], var2[0]
    inv = jax.lax.rsqrt(var + 0.001)
    scale = gamma * inv
    off = beta - mean * scale
    shiftq = jnp.concatenate(
        [jnp.zeros((1, IN), jnp.float32)]
        + [((c * var + mean) * gamma - beta)[None, :] for c in COEFS], axis=0)
    n1 = chosen_index[:, 1]
    q1 = chosen_index[:, 2]
    n2 = chosen_index[:, 3]
    q2 = chosen_index[:, 4]
    s1 = shiftq[q1, n1][None, :]
    s2 = shiftq[q2, n2][None, :]
    w_q = jnp.transpose(w[:IN * Q].reshape(IN, Q, OUT), (1, 0, 2))
    mx, min_x, output = _main(x, scale[None, :], off[None, :], shiftq,
                              n1[None, :], n2[None, :], w_q, s1, s2,
                              w[IN * Q:], biases.reshape(1, 1))
    max_x = jnp.transpose(mx, (1, 2, 0))
    return (output, w, max_x, min_x)


# speedup vs baseline: 1.2187x; 1.0112x over previous
import jax
import jax.numpy as jnp
from jax.experimental import pallas as pl
from jax.experimental.pallas import tpu as pltpu

B, IN, Q, M, OUT = 4096, 2048, 6, 512, 16
COEFS = (-3.0, -0.834, -0.248, 0.248, 0.834)

B_T = 128
NB = B // B_T


def _body(x_ref, gamma_ref, beta_ref, n1_ref, n2_ref, cq1_ref, cq2_ref,
          wq_ref, wmin_ref, b_ref,
          mx_ref, minx_ref, out_ref,
          sums_ref, stat_ref, svec_ref):
    ph = pl.program_id(0)
    ib = pl.program_id(1)

    def onehots(dtype):
        rows = jax.lax.broadcasted_iota(jnp.int32, (IN, M), 0)
        return ((rows == n1_ref[...]).astype(dtype),
                (rows == n2_ref[...]).astype(dtype))

    @pl.when(ph == 0)
    def _phase0():
        xb = x_ref[pl.ds(ib * B_T, B_T), :]
        s = jnp.sum(xb, axis=0)
        ss = jnp.sum(xb * xb, axis=0)
        first = ib == 0
        sums_ref[0, :] = jnp.where(first, s, sums_ref[0, :] + s)
        sums_ref[1, :] = jnp.where(first, ss, sums_ref[1, :] + ss)

    @pl.when((ph == 1) & (ib == 0))
    def _derive():
        mean = sums_ref[0, :] * (1.0 / B)
        var = sums_ref[1, :] * (1.0 / B) - mean * mean
        gamma = gamma_ref[0, :]
        beta = beta_ref[0, :]
        scale = gamma * jax.lax.rsqrt(var + 0.001)
        stat_ref[0, :] = scale
        stat_ref[1, :] = beta - mean * scale
        for qi, c in enumerate(COEFS):
            stat_ref[2 + qi, :] = (c * var + mean) * gamma - beta
        t = jnp.concatenate([var[None, :], mean[None, :],
                             gamma[None, :], beta[None, :]], axis=0)
        eqf1, eqf2 = onehots(jnp.float32)
        for k, (eqf, cq_ref) in enumerate(((eqf1, cq1_ref), (eqf2, cq2_ref))):
            tg = jnp.dot(t, eqf, preferred_element_type=jnp.float32)
            cq = cq_ref[0, :]
            sv = (cq * tg[0, :] + tg[1, :]) * tg[2, :] - tg[3, :]
            svec_ref[k, :] = jnp.where(cq == 0.0, 0.0, sv)

    @pl.when(ph == 1)
    def _phase1():
        scale = stat_ref[0, :]
        off = stat_ref[1, :]
        xb = x_ref[pl.ds(ib * B_T, B_T), :]
        normed = xb * scale + off
        nb = normed.astype(jnp.bfloat16)
        pacc = jnp.zeros((B_T, OUT), jnp.float32)
        for q in range(Q):
            if q == 0:
                rq = jnp.maximum(normed, 0.0)
            else:
                rq = jnp.maximum(normed - stat_ref[1 + q, :], 0.0)
            mx_ref[q] = rq
            pacc += jnp.dot(rq.astype(jnp.bfloat16),
                            wq_ref[q].astype(jnp.bfloat16),
                            preferred_element_type=jnp.float32)
        p1, p2 = onehots(jnp.bfloat16)
        g1 = jnp.dot(nb, p1, preferred_element_type=jnp.float32)
        g2 = jnp.dot(nb, p2, preferred_element_type=jnp.float32)
        d1 = jnp.maximum(g1 - svec_ref[0, :], 0.0)
        d2 = jnp.maximum(g2 - svec_ref[1, :], 0.0)
        mn = jnp.minimum(d1, d2)
        minx_ref[...] = mn
        out_ref[...] = (pacc
                        + jnp.dot(mn, wmin_ref[...],
                                  preferred_element_type=jnp.float32)
                        + b_ref[0, 0])


def _fused(x, gamma, beta, n1, n2, cq1, cq2, w_q, w_min, biases):
    return pl.pallas_call(
        _body,
        grid=(2, NB),
        in_specs=[
            pl.BlockSpec((B, IN), lambda ph, ib: (0, 0)),
            pl.BlockSpec((1, IN), lambda ph, ib: (0, 0)),
            pl.BlockSpec((1, IN), lambda ph, ib: (0, 0)),
            pl.BlockSpec((1, M), lambda ph, ib: (0, 0)),
            pl.BlockSpec((1, M), lambda ph, ib: (0, 0)),
            pl.BlockSpec((1, M), lambda ph, ib: (0, 0)),
            pl.BlockSpec((1, M), lambda ph, ib: (0, 0)),
            pl.BlockSpec((Q, IN, OUT), lambda ph, ib: (0, 0, 0)),
            pl.BlockSpec((M, OUT), lambda ph, ib: (0, 0)),
            pl.BlockSpec((1, 1), lambda ph, ib: (0, 0)),
        ],
        out_specs=[
            pl.BlockSpec((Q, B_T, IN), lambda ph, ib: (0, ib * ph, 0)),
            pl.BlockSpec((B_T, M), lambda ph, ib: (ib * ph, 0)),
            pl.BlockSpec((B_T, OUT), lambda ph, ib: (ib * ph, 0)),
        ],
        out_shape=[
            jax.ShapeDtypeStruct((Q, B, IN), jnp.float32),
            jax.ShapeDtypeStruct((B, M), jnp.float32),
            jax.ShapeDtypeStruct((B, OUT), jnp.float32),
        ],
        scratch_shapes=[
            pltpu.VMEM((2, IN), jnp.float32),
            pltpu.VMEM((8, IN), jnp.float32),
            pltpu.VMEM((2, M), jnp.float32),
        ],
        compiler_params=pltpu.CompilerParams(
            dimension_semantics=("arbitrary", "arbitrary"),
        ),
    )(x, gamma, beta, n1, n2, cq1, cq2, w_q, w_min, biases)


def kernel(x, init_struct, beta, gamma, w, biases, chosen_index):
    n1 = chosen_index[:, 1]
    q1 = chosen_index[:, 2]
    n2 = chosen_index[:, 3]
    q2 = chosen_index[:, 4]
    cvals = jnp.array((0.0,) + COEFS, jnp.float32)
    cq1 = cvals[q1]
    cq2 = cvals[q2]
    w_q = jnp.transpose(w[:IN * Q].reshape(IN, Q, OUT), (1, 0, 2))
    mx, min_x, output = _fused(x, gamma[None, :], beta[None, :],
                               n1[None, :], n2[None, :],
                               cq1[None, :], cq2[None, :],
                               w_q, w[IN * Q:], biases.reshape(1, 1))
    max_x = jnp.transpose(mx, (1, 2, 0))
    return (output, w, max_x, min_x)
